# baseline (device time: 9000 ns/iter reference)
import jax
import jax.numpy as jnp
from jax import lax
from jax.experimental import pallas as pl
from jax.experimental.pallas import tpu as pltpu

N_DEV = 8
_PLANE_MASKS = (1, 2, 3)


def kernel(x):
    m_per, n = x.shape

    def body(
        x_ref, out_ref,
        acc_ref, pair_ref, recv_ref,
        z_send_sem, z_recv_sem, send_sems, recv_sems, plane_ready,
    ):
        me = lax.axis_index("i")
        zp = me ^ 4
        my_slot = me & 3

        barrier_sem = pltpu.get_barrier_semaphore()
        pl.semaphore_signal(
            barrier_sem, inc=1,
            device_id=(zp,), device_id_type=pl.DeviceIdType.MESH,
        )
        for mask in _PLANE_MASKS:
            pl.semaphore_signal(
                plane_ready, inc=1,
                device_id=(me ^ mask,), device_id_type=pl.DeviceIdType.MESH,
            )

        acc_ref[:, :] = jnp.max(x_ref[:, :], axis=0, keepdims=True)

        pl.semaphore_wait(barrier_sem, 1)
        rdma_z = pltpu.make_async_remote_copy(
            src_ref=acc_ref,
            dst_ref=pair_ref,
            send_sem=z_send_sem,
            recv_sem=z_recv_sem,
            device_id=(zp,),
            device_id_type=pl.DeviceIdType.MESH,
        )
        rdma_z.start()
        rdma_z.wait_recv()
        pair_ref[:, :] = jnp.maximum(acc_ref[:, :], pair_ref[:, :])
        recv_ref[my_slot] = pair_ref[:, :]

        pl.semaphore_wait(plane_ready, len(_PLANE_MASKS))
        rdmas = []
        for mask in _PLANE_MASKS:
            tgt = me ^ mask
            rdma = pltpu.make_async_remote_copy(
                src_ref=pair_ref,
                dst_ref=recv_ref.at[my_slot],
                send_sem=send_sems.at[tgt & 3],
                recv_sem=recv_sems.at[my_slot],
                device_id=(tgt,),
                device_id_type=pl.DeviceIdType.MESH,
            )
            rdma.start()
            rdmas.append(rdma)

        for mask in _PLANE_MASKS:
            src = me ^ mask
            pltpu.make_async_remote_copy(
                src_ref=pair_ref,
                dst_ref=recv_ref.at[src & 3],
                send_sem=send_sems.at[src & 3],
                recv_sem=recv_sems.at[src & 3],
                device_id=(src,),
                device_id_type=pl.DeviceIdType.MESH,
            ).wait_recv()

        out_ref[:, :] = jnp.max(recv_ref[:, :, :], axis=0)

        rdma_z.wait_send()
        for rdma in rdmas:
            rdma.wait_send()

    return pl.pallas_call(
        body,
        out_shape=jax.ShapeDtypeStruct((1, n), jnp.float32),
        in_specs=[pl.BlockSpec(memory_space=pltpu.VMEM)],
        out_specs=pl.BlockSpec(memory_space=pltpu.VMEM),
        scratch_shapes=[
            pltpu.VMEM((1, n), jnp.float32),
            pltpu.VMEM((1, n), jnp.float32),
            pltpu.VMEM((4, 1, n), jnp.float32),
            pltpu.SemaphoreType.DMA,
            pltpu.SemaphoreType.DMA,
            pltpu.SemaphoreType.DMA((4,)),
            pltpu.SemaphoreType.DMA((4,)),
            pltpu.SemaphoreType.REGULAR,
        ],
        compiler_params=pltpu.CompilerParams(collective_id=0),
    )(x)


# device time: 8049 ns/iter; 1.1182x vs baseline; 1.1182x over previous
import jax
import jax.numpy as jnp
from jax import lax
from jax.experimental import pallas as pl
from jax.experimental.pallas import tpu as pltpu

N_DEV = 8


def kernel(x):
    m_per, n = x.shape

    def body(x_ref, out_ref, acc_ref, recv_ref, send_sems, recv_sems):
        my_pos = lax.axis_index("i")

        barrier_sem = pltpu.get_barrier_semaphore()
        for off in range(1, N_DEV):
            pl.semaphore_signal(
                barrier_sem, inc=1,
                device_id=((my_pos + off) % N_DEV,),
                device_id_type=pl.DeviceIdType.MESH,
            )

        acc_ref[:, :] = jnp.max(x_ref[:, :], axis=0, keepdims=True)
        recv_ref[my_pos] = acc_ref[:, :]

        pl.semaphore_wait(barrier_sem, N_DEV - 1)

        rdmas = []
        for off in range(1, N_DEV):
            tgt = (my_pos + off) % N_DEV
            rdma = pltpu.make_async_remote_copy(
                src_ref=acc_ref,
                dst_ref=recv_ref.at[my_pos],
                send_sem=send_sems.at[tgt],
                recv_sem=recv_sems.at[my_pos],
                device_id=(tgt,),
                device_id_type=pl.DeviceIdType.MESH,
            )
            rdma.start()
            rdmas.append(rdma)

        for off in range(1, N_DEV):
            src = (my_pos + off) % N_DEV
            pltpu.make_async_remote_copy(
                src_ref=acc_ref,
                dst_ref=recv_ref.at[src],
                send_sem=send_sems.at[src],
                recv_sem=recv_sems.at[src],
                device_id=(src,),
                device_id_type=pl.DeviceIdType.MESH,
            ).wait_recv()

        out_ref[:, :] = jnp.max(recv_ref[:, :, :], axis=0)

        for rdma in rdmas:
            rdma.wait_send()

    return pl.pallas_call(
        body,
        out_shape=jax.ShapeDtypeStruct((1, n), jnp.float32),
        in_specs=[pl.BlockSpec(memory_space=pltpu.VMEM)],
        out_specs=pl.BlockSpec(memory_space=pltpu.VMEM),
        scratch_shapes=[
            pltpu.VMEM((1, n), jnp.float32),
            pltpu.VMEM((N_DEV, 1, n), jnp.float32),
            pltpu.SemaphoreType.DMA((N_DEV,)),
            pltpu.SemaphoreType.DMA((N_DEV,)),
        ],
        compiler_params=pltpu.CompilerParams(collective_id=0),
    )(x)
